# baseline (speedup 1.0000x reference)
import functools

import jax
import jax.numpy as jnp
from jax import lax
from jax.experimental import pallas as pl
from jax.experimental.pallas import tpu as pltpu
from jax.experimental.pallas import tpu_sc as plsc

B = 16384
EMB = 128
EPS = 1e-5

NUM_WORKERS = 32
ROWS_PER_W = B // NUM_WORKERS
CHUNK = 128
NCHUNK = ROWS_PER_W // CHUNK


def _sc_gather_body(uidx_hbm, iidx_hbm, utab_hbm, itab_hbm,
                    ue_out, ie_out, idx_v, rows_v, sem):
    wid = lax.axis_index("s") * 2 + lax.axis_index("c")
    base = wid * ROWS_PER_W
    for idx_hbm, tab_hbm, out_hbm in ((uidx_hbm, utab_hbm, ue_out),
                                      (iidx_hbm, itab_hbm, ie_out)):
        pltpu.sync_copy(idx_hbm.at[pl.ds(base, ROWS_PER_W)], idx_v)
        copies = []
        for j in range(NCHUNK):
            copies.append(pltpu.async_copy(
                tab_hbm.at[idx_v.at[pl.ds(j * CHUNK, CHUNK)]],
                rows_v.at[pl.ds(j * CHUNK, CHUNK)], sem))
        for c in copies:
            c.wait()
        pltpu.sync_copy(rows_v, out_hbm.at[pl.ds(base, ROWS_PER_W)])


@functools.cache
def _make_gather():
    return pl.kernel(
        _sc_gather_body,
        mesh=plsc.VectorSubcoreMesh(core_axis_name="c", subcore_axis_name="s"),
        out_type=[jax.ShapeDtypeStruct((B, EMB), jnp.float32),
                  jax.ShapeDtypeStruct((B, EMB), jnp.float32)],
        scratch_types=[pltpu.VMEM((ROWS_PER_W,), jnp.int32),
                       pltpu.VMEM((ROWS_PER_W, EMB), jnp.float32),
                       pltpu.SemaphoreType.DMA],
    )


def _bn_relu(x, g, beta):
    mu = jnp.mean(x, axis=0, keepdims=True)
    var = jnp.mean((x - mu) ** 2, axis=0, keepdims=True)
    return jnp.maximum(g * (x - mu) * lax.rsqrt(var + EPS) + beta, 0.0)


def _mlp_body(ue, ie, uW1, ub1, ug1, ubeta1, uW2, ub2,
              iW1, ib1, ig1, ibeta1, iW2, ib2,
              W3, b3, g3, beta3, Wo, bo, out):
    P = lax.Precision.DEFAULT
    x = jnp.dot(ue[...], uW1[...], precision=P) + ub1[...]
    x = _bn_relu(x, ug1[...], ubeta1[...])
    u = jnp.dot(x, uW2[...], precision=P) + ub2[...]
    y = jnp.dot(ie[...], iW1[...], precision=P) + ib1[...]
    y = _bn_relu(y, ig1[...], ibeta1[...])
    it = jnp.dot(y, iW2[...], precision=P) + ib2[...]
    comb = jnp.concatenate([u, it], axis=1)
    h = jnp.dot(comb, W3[...], precision=P) + b3[...]
    h = _bn_relu(h, g3[...], beta3[...])
    out[...] = jnp.dot(h, Wo[...], precision=P) + bo[...]


_mlp = pl.pallas_call(
    _mlp_body,
    out_shape=jax.ShapeDtypeStruct((B, 1), jnp.float32),
)


def kernel(user_input, item_input, user_table, item_table,
           uW1, ub1, ug1, ubeta1, uW2, ub2,
           iW1, ib1, ig1, ibeta1, iW2, ib2,
           W3, b3, g3, beta3, Wo, bo):
    uidx = user_input.astype(jnp.int32)
    iidx = item_input.astype(jnp.int32)
    ue, ie = _make_gather()(uidx, iidx, user_table, item_table)
    r = lambda v: v.reshape(1, -1)
    return _mlp(ue, ie, uW1, r(ub1), r(ug1), r(ubeta1), uW2, r(ub2),
                iW1, r(ib1), r(ig1), r(ibeta1), iW2, r(ib2),
                W3, r(b3), r(g3), r(beta3), Wo, r(bo))


# speedup vs baseline: 1.1695x; 1.1695x over previous
import functools

import jax
import jax.numpy as jnp
from jax import lax
from jax.experimental import pallas as pl
from jax.experimental.pallas import tpu as pltpu
from jax.experimental.pallas import tpu_sc as plsc

B = 16384
EMB = 128
EPS = 1e-5

NUM_WORKERS = 32
ROWS_PER_W = B // NUM_WORKERS
CHUNK = 128
NCHUNK = ROWS_PER_W // CHUNK


def _sc_gather_body(uidx_hbm, iidx_hbm, utab_hbm, itab_hbm,
                    x_out, idx_v, rows_v, sem):
    wid = lax.axis_index("s") * 2 + lax.axis_index("c")
    base = wid * ROWS_PER_W
    for idx_hbm, tab_hbm, col in ((uidx_hbm, utab_hbm, 0),
                                  (iidx_hbm, itab_hbm, EMB)):
        pltpu.sync_copy(idx_hbm.at[pl.ds(base, ROWS_PER_W)], idx_v)
        copies = []
        for j in range(NCHUNK):
            copies.append(pltpu.async_copy(
                tab_hbm.at[idx_v.at[pl.ds(j * CHUNK, CHUNK)]],
                rows_v.at[pl.ds(j * CHUNK, CHUNK)], sem))
        for c in copies:
            c.wait()
        pltpu.sync_copy(rows_v,
                        x_out.at[pl.ds(base, ROWS_PER_W), pl.ds(col, EMB)])


@functools.cache
def _make_gather():
    return pl.kernel(
        _sc_gather_body,
        mesh=plsc.VectorSubcoreMesh(core_axis_name="c", subcore_axis_name="s"),
        out_type=jax.ShapeDtypeStruct((B, 2 * EMB), jnp.float32),
        scratch_types=[pltpu.VMEM((ROWS_PER_W,), jnp.int32),
                       pltpu.VMEM((ROWS_PER_W, EMB), jnp.float32),
                       pltpu.SemaphoreType.DMA],
    )


def _bn_fold(x, g, beta):
    mu = jnp.mean(x, axis=0, keepdims=True)
    var = jnp.mean(x * x, axis=0, keepdims=True) - mu * mu
    a = g * lax.rsqrt(var + EPS)
    c = beta - a * mu
    return jnp.maximum(a * x + c, 0.0)


def _mlp_body(x2, W1, b1, g1, beta1, W23, b23, g3, beta3, Wo, bo, out):
    x = jnp.dot(x2[...], W1[...]) + b1[...]
    y = _bn_fold(x, g1[...], beta1[...])
    h = jnp.dot(y, W23[...]) + b23[...]
    hh = _bn_fold(h, g3[...], beta3[...])
    out[...] = jnp.dot(hh, Wo[...]) + bo[...]


_mlp = pl.pallas_call(
    _mlp_body,
    out_shape=jax.ShapeDtypeStruct((B, 1), jnp.float32),
)


def kernel(user_input, item_input, user_table, item_table,
           uW1, ub1, ug1, ubeta1, uW2, ub2,
           iW1, ib1, ig1, ibeta1, iW2, ib2,
           W3, b3, g3, beta3, Wo, bo):
    uidx = user_input.astype(jnp.int32)
    iidx = item_input.astype(jnp.int32)
    x2 = _make_gather()(uidx, iidx, user_table, item_table)

    W1 = jnp.concatenate(
        [jnp.concatenate([uW1, jnp.zeros_like(uW1)], axis=1),
         jnp.concatenate([jnp.zeros_like(iW1), iW1], axis=1)], axis=0)
    W2 = jnp.concatenate(
        [jnp.concatenate([uW2, jnp.zeros_like(uW2)], axis=1),
         jnp.concatenate([jnp.zeros_like(iW2), iW2], axis=1)], axis=0)
    W23 = W2 @ W3
    b23 = jnp.concatenate([ub2, ib2]) @ W3 + b3
    r = lambda v: v.reshape(1, -1)
    return _mlp(x2, W1, r(jnp.concatenate([ub1, ib1])),
                r(jnp.concatenate([ug1, ig1])),
                r(jnp.concatenate([ubeta1, ibeta1])),
                W23, r(b23), r(g3), r(beta3), Wo, r(bo))


# speedup vs baseline: 1.2348x; 1.0558x over previous
import functools

import jax
import jax.numpy as jnp
from jax import lax
from jax.experimental import pallas as pl
from jax.experimental.pallas import tpu as pltpu
from jax.experimental.pallas import tpu_sc as plsc

B = 16384
EMB = 128
EPS = 1e-5

NUM_WORKERS = 32
ROWS_PER_W = B // NUM_WORKERS
CHUNK = 128
NCHUNK = ROWS_PER_W // CHUNK


def _sc_gather_body(uidx_hbm, iidx_hbm, utab_hbm, itab_hbm,
                    x_out, idx_v, rows_v, sem):
    wid = lax.axis_index("s") * 2 + lax.axis_index("c")
    base = wid * ROWS_PER_W
    for idx_hbm, tab_hbm, col in ((uidx_hbm, utab_hbm, 0),
                                  (iidx_hbm, itab_hbm, EMB)):
        pltpu.sync_copy(idx_hbm.at[pl.ds(base, ROWS_PER_W)], idx_v)
        copies = []
        for j in range(NCHUNK):
            copies.append(pltpu.async_copy(
                tab_hbm.at[idx_v.at[pl.ds(j * CHUNK, CHUNK)]],
                rows_v.at[pl.ds(j * CHUNK, CHUNK)], sem))
        for c in copies:
            c.wait()
        pltpu.sync_copy(rows_v,
                        x_out.at[pl.ds(base, ROWS_PER_W), pl.ds(col, EMB)])


@functools.cache
def _make_gather():
    return pl.kernel(
        _sc_gather_body,
        mesh=plsc.VectorSubcoreMesh(core_axis_name="c", subcore_axis_name="s"),
        out_type=jax.ShapeDtypeStruct((B, 2 * EMB), jnp.float32),
        scratch_types=[pltpu.VMEM((ROWS_PER_W,), jnp.int32),
                       pltpu.VMEM((ROWS_PER_W, EMB), jnp.float32),
                       pltpu.SemaphoreType.DMA],
    )


def _bn_fold(x, g, beta):
    mu = jnp.mean(x, axis=0, keepdims=True)
    var = jnp.mean(x * x, axis=0, keepdims=True) - mu * mu
    a = g * lax.rsqrt(var + EPS)
    c = beta - a * mu
    return jnp.maximum(a * x + c, 0.0)


def _mlp_body(x2, W1, b1, g1, beta1, W23, b23, g3, beta3, Wo, bo, out):
    x = jnp.dot(x2[...], W1[...]) + b1[...]
    y = _bn_fold(x, g1[...], beta1[...])
    h = jnp.dot(y, W23[...]) + b23[...]
    hh = _bn_fold(h, g3[...], beta3[...])
    out[...] = (jnp.dot(hh, Wo[...]) + bo[...]).reshape(B)


_mlp = pl.pallas_call(
    _mlp_body,
    out_shape=jax.ShapeDtypeStruct((B,), jnp.float32),
)


def _mlp_out_2d(*args):
    return _mlp(*args).reshape(B, 1)


def kernel(user_input, item_input, user_table, item_table,
           uW1, ub1, ug1, ubeta1, uW2, ub2,
           iW1, ib1, ig1, ibeta1, iW2, ib2,
           W3, b3, g3, beta3, Wo, bo):
    uidx = user_input.astype(jnp.int32)
    iidx = item_input.astype(jnp.int32)
    x2 = _make_gather()(uidx, iidx, user_table, item_table)

    W1 = jnp.concatenate(
        [jnp.concatenate([uW1, jnp.zeros_like(uW1)], axis=1),
         jnp.concatenate([jnp.zeros_like(iW1), iW1], axis=1)], axis=0)
    W2 = jnp.concatenate(
        [jnp.concatenate([uW2, jnp.zeros_like(uW2)], axis=1),
         jnp.concatenate([jnp.zeros_like(iW2), iW2], axis=1)], axis=0)
    W23 = W2 @ W3
    b23 = jnp.concatenate([ub2, ib2]) @ W3 + b3
    r = lambda v: v.reshape(1, -1)
    return _mlp_out_2d(x2, W1, r(jnp.concatenate([ub1, ib1])),
                r(jnp.concatenate([ug1, ig1])),
                r(jnp.concatenate([ubeta1, ibeta1])),
                W23, r(b23), r(g3), r(beta3), Wo, r(bo))
